# baseline (device time: 27461 ns/iter reference)
import jax
import jax.numpy as jnp
from jax import lax
from jax.experimental import pallas as pl
from jax.experimental.pallas import tpu as pltpu

N_DEV = 32
COL_TILE = 256


def kernel(x):
    m_per, n_per = x.shape

    def body(x_ref, o_ref, stats_ref, send_sems, recv_sems):
        me = lax.axis_index("i")

        barrier_sem = pltpu.get_barrier_semaphore()
        for d in (1, N_DEV - 1):
            pl.semaphore_signal(
                barrier_sem, inc=1,
                device_id=((me + d) % N_DEV,),
                device_id_type=pl.DeviceIdType.MESH,
            )

        xv = x_ref[:, :]
        m_run = jnp.max(xv[:, :COL_TILE], axis=1)
        s_run = jnp.sum(jnp.exp(xv[:, :COL_TILE] - m_run[:, None]), axis=1)
        for t in range(1, n_per // COL_TILE):
            tile = xv[:, t * COL_TILE:(t + 1) * COL_TILE]
            m_new = jnp.maximum(m_run, jnp.max(tile, axis=1))
            s_run = s_run * jnp.exp(m_run - m_new) + jnp.sum(
                jnp.exp(tile - m_new[:, None]), axis=1)
            m_run = m_new
        stats_ref[me, pl.ds(0, m_per)] = m_run
        stats_ref[me, pl.ds(m_per, m_per)] = s_run

        pl.semaphore_wait(barrier_sem, 2)

        rdmas = []
        for k in sorted(range(1, N_DEV), key=lambda k: -min(k, N_DEV - k)):
            rdma = pltpu.make_async_remote_copy(
                src_ref=stats_ref.at[me],
                dst_ref=stats_ref.at[me],
                send_sem=send_sems.at[k - 1],
                recv_sem=recv_sems.at[k - 1],
                device_id=((me + k) % N_DEV,),
                device_id_type=pl.DeviceIdType.MESH,
            )
            rdma.start()
            rdmas.append(rdma)

        for rdma in rdmas:
            rdma.wait_recv()

        all_m = stats_ref[:, pl.ds(0, m_per)]
        all_s = stats_ref[:, pl.ds(m_per, m_per)]
        m_g = jnp.max(all_m, axis=0)
        s_g = jnp.sum(all_s * jnp.exp(all_m - m_g[None, :]), axis=0)
        r = 1.0 / s_g
        o_ref[:, :] = jnp.exp(xv - m_g[:, None]) * r[:, None]

        for rdma in rdmas:
            rdma.wait_send()

    return pl.pallas_call(
        body,
        out_shape=jax.ShapeDtypeStruct((m_per, n_per), jnp.float32),
        in_specs=[pl.BlockSpec(memory_space=pltpu.VMEM)],
        out_specs=pl.BlockSpec(memory_space=pltpu.VMEM),
        scratch_shapes=[
            pltpu.VMEM((N_DEV, 2 * m_per), jnp.float32),
            pltpu.SemaphoreType.DMA((N_DEV - 1,)),
            pltpu.SemaphoreType.DMA((N_DEV - 1,)),
        ],
        compiler_params=pltpu.CompilerParams(collective_id=0),
    )(x)


# device time: 24452 ns/iter; 1.1231x vs baseline; 1.1231x over previous
import jax
import jax.numpy as jnp
from jax import lax
from jax.experimental import pallas as pl
from jax.experimental.pallas import tpu as pltpu

N_DEV = 32
COL_TILE = 256


def kernel(x):
    m_per, n_per = x.shape

    def body(x_ref, o_ref, stats_ref, send_sems, recv_sems):
        me = lax.axis_index("i")

        barrier_sem = pltpu.get_barrier_semaphore()
        for k in range(1, N_DEV):
            pl.semaphore_signal(
                barrier_sem, inc=1,
                device_id=((me + k) % N_DEV,),
                device_id_type=pl.DeviceIdType.MESH,
            )

        xv = x_ref[:, :]
        m_run = jnp.max(xv[:, :COL_TILE], axis=1)
        s_run = jnp.sum(jnp.exp(xv[:, :COL_TILE] - m_run[:, None]), axis=1)
        for t in range(1, n_per // COL_TILE):
            tile = xv[:, t * COL_TILE:(t + 1) * COL_TILE]
            m_new = jnp.maximum(m_run, jnp.max(tile, axis=1))
            s_run = s_run * jnp.exp(m_run - m_new) + jnp.sum(
                jnp.exp(tile - m_new[:, None]), axis=1)
            m_run = m_new
        stats_ref[me, pl.ds(0, m_per)] = m_run
        stats_ref[me, pl.ds(m_per, m_per)] = s_run

        pl.semaphore_wait(barrier_sem, N_DEV - 1)

        rdmas = []
        for k in range(1, N_DEV):
            rdma = pltpu.make_async_remote_copy(
                src_ref=stats_ref.at[me],
                dst_ref=stats_ref.at[me],
                send_sem=send_sems.at[k - 1],
                recv_sem=recv_sems.at[k - 1],
                device_id=((me + k) % N_DEV,),
                device_id_type=pl.DeviceIdType.MESH,
            )
            rdma.start()
            rdmas.append(rdma)

        for rdma in rdmas:
            rdma.wait_recv()

        all_m = stats_ref[:, pl.ds(0, m_per)]
        all_s = stats_ref[:, pl.ds(m_per, m_per)]
        m_g = jnp.max(all_m, axis=0)
        s_g = jnp.sum(all_s * jnp.exp(all_m - m_g[None, :]), axis=0)
        r = 1.0 / s_g
        o_ref[:, :] = jnp.exp(xv - m_g[:, None]) * r[:, None]

        for rdma in rdmas:
            rdma.wait_send()

    return pl.pallas_call(
        body,
        out_shape=jax.ShapeDtypeStruct((m_per, n_per), jnp.float32),
        in_specs=[pl.BlockSpec(memory_space=pltpu.VMEM)],
        out_specs=pl.BlockSpec(memory_space=pltpu.VMEM),
        scratch_shapes=[
            pltpu.VMEM((N_DEV, 2 * m_per), jnp.float32),
            pltpu.SemaphoreType.DMA((N_DEV - 1,)),
            pltpu.SemaphoreType.DMA((N_DEV - 1,)),
        ],
        compiler_params=pltpu.CompilerParams(collective_id=0),
    )(x)
